# Initial kernel scaffold; baseline (speedup 1.0000x reference)
import jax
import jax.numpy as jnp
from jax.experimental import pallas as pl


def kernel(feature_1, feature_2_states, feature_3_states, feature_4, feature_5_states, feature_6, edge_index_1, edge_index_2, params):
    raise NotImplementedError("write your pallas kernel here")


# speedup vs baseline: 4.9326x; 4.9326x over previous
import functools

import jax
import jax.numpy as jnp
from jax import lax
from jax.experimental import pallas as pl
from jax.experimental.pallas import tpu as pltpu
from jax.experimental.pallas import tpu_sc as plsc

_N = 10000
_E = 320000
_T = 20
_D = 128
_H3 = 192
_NC = 2
_NS = 16
_NPAD = 10112
_RPT = _NPAD // _NS
_CH = 128
_EPAD = 327680
_EPT_DEG = _EPAD // (_NC * _NS)
_EPT_SC = _EPAD // _NS

_BG = 1000
_BL = 1000


def _gru_body(x_ref, wr, wz, wn, ur, uz, un, br, bz, bn, bhn, out_ref):
    B = x_ref.shape[0]
    h = jnp.zeros((B, _H3), jnp.float32)
    for t in range(_T):
        xt = x_ref[:, t, :]
        r = jax.nn.sigmoid(
            jnp.dot(xt, wr[...], preferred_element_type=jnp.float32)
            + jnp.dot(h, ur[...], preferred_element_type=jnp.float32) + br[...])
        z = jax.nn.sigmoid(
            jnp.dot(xt, wz[...], preferred_element_type=jnp.float32)
            + jnp.dot(h, uz[...], preferred_element_type=jnp.float32) + bz[...])
        n = jnp.tanh(
            jnp.dot(xt, wn[...], preferred_element_type=jnp.float32) + bn[...]
            + r * (jnp.dot(h, un[...], preferred_element_type=jnp.float32) + bhn[...]))
        h = (1.0 - z) * n + z * h
    out_ref[...] = h


def _gru(x, wr, wz, wn, ur, uz, un, br, bz, bn, bhn):
    grid = _N // _BG
    wspec = pl.BlockSpec((_D, _H3), lambda i: (0, 0))
    uspec = pl.BlockSpec((_H3, _H3), lambda i: (0, 0))
    bspec = pl.BlockSpec((1, _H3), lambda i: (0, 0))
    return pl.pallas_call(
        _gru_body,
        grid=(grid,),
        in_specs=[pl.BlockSpec((_BG, _T, _D), lambda i: (i, 0, 0)),
                  wspec, wspec, wspec, uspec, uspec, uspec,
                  bspec, bspec, bspec, bspec],
        out_specs=pl.BlockSpec((_BG, _H3), lambda i: (i, 0)),
        out_shape=jax.ShapeDtypeStruct((_N, _H3), jnp.float32),
    )(x, wr, wz, wn, ur, uz, un, br, bz, bn, bhn)


def _dinv(dga_ref, dgb_ref):
    deg = dga_ref[:, 0:1] + dgb_ref[:, 0:1] + 1.0
    return lax.rsqrt(deg)


def _split_out(hs, outa, outb):
    B = hs.shape[0]
    outa[0] = hs[:, :128]
    outb[0] = jnp.concatenate(
        [hs[:, 128:], jnp.zeros((B, 64), jnp.float32)], axis=1)


def _join(a_ref, b_ref):
    return jnp.concatenate([a_ref[0], b_ref[0][:, :64]], axis=1)


def _layer0_body(x_ref, a_ref, c_ref, dga_ref, dgb_ref, outa, outb):
    di = _dinv(dga_ref, dgb_ref)
    h = jnp.dot(x_ref[...], a_ref[...],
                preferred_element_type=jnp.float32) + c_ref[...]
    _split_out(h * di, outa, outb)


def _hs_specs():
    return [pl.BlockSpec((1, _BL, 128), lambda i: (0, i, 0)),
            pl.BlockSpec((1, _BL, 128), lambda i: (0, i, 0))]


def _hs_shapes():
    return [jax.ShapeDtypeStruct((1, _N, 128), jnp.float32),
            jax.ShapeDtypeStruct((1, _N, 128), jnp.float32)]


def _layer0(x, a, c, dga, dgb):
    K = x.shape[1]
    grid = _N // _BL
    outa, outb = pl.pallas_call(
        _layer0_body,
        grid=(grid,),
        in_specs=[pl.BlockSpec((_BL, K), lambda i: (i, 0)),
                  pl.BlockSpec((K, _H3), lambda i: (0, 0)),
                  pl.BlockSpec((1, _H3), lambda i: (0, 0)),
                  pl.BlockSpec((_BL, 128), lambda i: (i, 0)),
                  pl.BlockSpec((_BL, 128), lambda i: (i, 0))],
        out_specs=_hs_specs(),
        out_shape=_hs_shapes(),
    )(x, a, c, dga, dgb)
    return jnp.concatenate([outa, outb], axis=0)


def _layer1_body(aca_ref, acb_ref, hsa_ref, hsb_ref, dga_ref, dgb_ref,
                 b_ref, w_ref, outa, outb):
    di = _dinv(dga_ref, dgb_ref)
    x = di * (_join(aca_ref, acb_ref) + _join(hsa_ref, hsb_ref)) + b_ref[...]
    x = jnp.maximum(x, 0.0)
    h = jnp.dot(x, w_ref[...], preferred_element_type=jnp.float32)
    _split_out(h * di, outa, outb)


def _acc_specs():
    return [pl.BlockSpec((1, _BL, 128), lambda i: (0, i, 0)),
            pl.BlockSpec((1, _BL, 128), lambda i: (1, i, 0))]


def _layer1(acc, hs, dga, dgb, b, w):
    grid = _N // _BL
    outa, outb = pl.pallas_call(
        _layer1_body,
        grid=(grid,),
        in_specs=_acc_specs() + _acc_specs() + [
            pl.BlockSpec((_BL, 128), lambda i: (i, 0)),
            pl.BlockSpec((_BL, 128), lambda i: (i, 0)),
            pl.BlockSpec((1, _H3), lambda i: (0, 0)),
            pl.BlockSpec((_H3, _H3), lambda i: (0, 0))],
        out_specs=_hs_specs(),
        out_shape=_hs_shapes(),
    )(acc, acc, hs, hs, dga, dgb, b, w)
    return jnp.concatenate([outa, outb], axis=0)


def _final_body(aca_ref, acb_ref, hsa_ref, hsb_ref, dga_ref, dgb_ref,
                b_ref, w2_ref, b2_ref, out_ref):
    di = _dinv(dga_ref, dgb_ref)
    x = di * (_join(aca_ref, acb_ref) + _join(hsa_ref, hsb_ref)) + b_ref[...]
    z = jnp.dot(x, w2_ref[...], preferred_element_type=jnp.float32) + b2_ref[...]
    m = jnp.max(z, axis=1, keepdims=True)
    e = jnp.exp(z - m)
    out_ref[...] = e / jnp.sum(e, axis=1, keepdims=True)


def _final(acc, hs, dga, dgb, b, w2, b2):
    grid = _N // _BL
    return pl.pallas_call(
        _final_body,
        grid=(grid,),
        in_specs=_acc_specs() + _acc_specs() + [
            pl.BlockSpec((_BL, 128), lambda i: (i, 0)),
            pl.BlockSpec((_BL, 128), lambda i: (i, 0)),
            pl.BlockSpec((1, _H3), lambda i: (0, 0)),
            pl.BlockSpec((_H3, 128), lambda i: (0, 0)),
            pl.BlockSpec((1, 128), lambda i: (0, 0))],
        out_specs=pl.BlockSpec((_BL, 128), lambda i: (i, 0)),
        out_shape=jax.ShapeDtypeStruct((_N, 128), jnp.float32),
    )(acc, acc, hs, hs, dga, dgb, b, w2, b2)


def _sc_mesh():
    return plsc.VectorSubcoreMesh(core_axis_name="c", subcore_axis_name="s")


def _sc_degree(dst_pad, zeros128, ones128):
    @functools.partial(
        pl.kernel,
        out_type=jax.ShapeDtypeStruct((_NC, _NPAD, 128), jnp.float32),
        mesh=_sc_mesh(),
        scratch_types=[
            pltpu.VMEM((_CH,), jnp.int32),
            pltpu.VMEM((_CH, 128), jnp.float32),
            pltpu.VMEM_SHARED((_NPAD, 128), jnp.float32),
        ],
    )
    def k(dst_hbm, zeros_hbm, ones_hbm, out_hbm, idx_v, ones_v, acc_sh):
        c = lax.axis_index("c")
        s = lax.axis_index("s")
        r0 = s * _RPT
        pltpu.sync_copy(zeros_hbm.at[pl.ds(r0, _RPT)],
                        acc_sh.at[pl.ds(r0, _RPT)])
        pltpu.sync_copy(ones_hbm, ones_v)
        plsc.subcore_barrier()
        base = (c * _NS + s) * _EPT_DEG

        def body(i, carry):
            off = pl.multiple_of(base + i * _CH, _CH)
            pltpu.sync_copy(dst_hbm.at[pl.ds(off, _CH)], idx_v)
            pltpu.sync_copy(ones_v, acc_sh.at[idx_v], add=True)
            return carry

        lax.fori_loop(0, _EPT_DEG // _CH, body, 0)
        plsc.subcore_barrier()
        pltpu.sync_copy(acc_sh.at[pl.ds(r0, _RPT)],
                        out_hbm.at[c].at[pl.ds(r0, _RPT)])

    return k(dst_pad, zeros128, ones128)


def _sc_scatter(hs, src_pad, dst_pad, zeros128):
    @functools.partial(
        pl.kernel,
        out_type=jax.ShapeDtypeStruct((_NC, _NPAD, 128), jnp.float32),
        mesh=_sc_mesh(),
        scratch_types=[
            pltpu.VMEM((_CH,), jnp.int32),
            pltpu.VMEM((_CH,), jnp.int32),
            pltpu.VMEM((_CH, 128), jnp.float32),
            pltpu.VMEM_SHARED((_NPAD, 128), jnp.float32),
            pltpu.SemaphoreType.DMA,
        ],
    )
    def k(hs_hbm, src_hbm, dst_hbm, zeros_hbm, out_hbm,
          src_v, dst_v, rows_v, acc_sh, sem):
        c = lax.axis_index("c")
        s = lax.axis_index("s")
        r0 = s * _RPT
        pltpu.sync_copy(zeros_hbm.at[pl.ds(r0, _RPT)],
                        acc_sh.at[pl.ds(r0, _RPT)])
        plsc.subcore_barrier()
        base = s * _EPT_SC
        half = hs_hbm.at[c]

        def body(i, carry):
            off = pl.multiple_of(base + i * _CH, _CH)
            pltpu.sync_copy(src_hbm.at[pl.ds(off, _CH)], src_v)
            pltpu.sync_copy(dst_hbm.at[pl.ds(off, _CH)], dst_v)
            pltpu.async_copy(half.at[src_v], rows_v, sem).wait()
            pltpu.sync_copy(rows_v, acc_sh.at[dst_v], add=True)
            return carry

        lax.fori_loop(0, _EPT_SC // _CH, body, 0)
        plsc.subcore_barrier()
        pltpu.sync_copy(acc_sh.at[pl.ds(r0, _RPT)],
                        out_hbm.at[c].at[pl.ds(r0, _RPT)])

    return k(hs, src_pad, dst_pad, zeros128)


def _gru_prep(Wi, Wh, bi, bh):
    WiT = Wi.T
    WhT = Wh.T
    wr, wz, wn = WiT[:, :_H3], WiT[:, _H3:2 * _H3], WiT[:, 2 * _H3:]
    ur, uz, un = WhT[:, :_H3], WhT[:, _H3:2 * _H3], WhT[:, 2 * _H3:]
    br = (bi[:_H3] + bh[:_H3])[None, :]
    bz = (bi[_H3:2 * _H3] + bh[_H3:2 * _H3])[None, :]
    bn = bi[2 * _H3:][None, :]
    bhn = bh[2 * _H3:][None, :]
    return wr, wz, wn, ur, uz, un, br, bz, bn, bhn


def _pad_edges(ei):
    src, dst = ei[0], ei[1]
    npad = _EPAD - _E
    src_pad = jnp.concatenate([src, jnp.zeros((npad,), jnp.int32)])
    dst_pad = jnp.concatenate([dst, jnp.full((npad,), _N, jnp.int32)])
    return src_pad, dst_pad


def _graph_side(xcat, R, bcat, ei, W0, b0, W1, b1, Wf, bf, zeros128, ones128):
    src_pad, dst_pad = _pad_edges(ei)
    degp = _sc_degree(dst_pad, zeros128, ones128)
    dga, dgb = degp[0, :_N], degp[1, :_N]

    A = jnp.dot(R.T, W0.T)
    c = jnp.dot(bcat, W0.T)[None, :]
    hs0 = _layer0(xcat, A, c, dga, dgb)
    acc0 = _sc_scatter(hs0, src_pad, dst_pad, zeros128)[:, :_N]
    hs1 = _layer1(acc0, hs0, dga, dgb, b0[None, :], W1.T)
    acc1 = _sc_scatter(hs1, src_pad, dst_pad, zeros128)[:, :_N]

    w2 = jnp.zeros((_H3, 128), jnp.float32).at[:, :2].set(Wf.T)
    b2 = jnp.full((128,), -1e30, jnp.float32).at[:2].set(bf)[None, :]
    out = _final(acc1, hs1, dga, dgb, b1[None, :], w2, b2)
    return out[:, :2]


def kernel(feature_1, feature_2_states, feature_3_states, feature_4,
           feature_5_states, feature_6, edge_index_1, edge_index_2, params):
    p = params
    fv1 = _gru(feature_1, *_gru_prep(p['gru1_Wi'], p['gru1_Wh'],
                                     p['gru1_bi'], p['gru1_bh']))
    fv6 = _gru(feature_6, *_gru_prep(p['gru2_Wi'], p['gru2_Wh'],
                                     p['gru2_bi'], p['gru2_bh']))

    z64 = jnp.zeros((64, _H3), jnp.float32)
    Rm = jnp.concatenate([
        jnp.concatenate([p['r1_W'], z64, z64], axis=1),
        jnp.concatenate([z64, p['r2_W'], z64], axis=1),
        jnp.concatenate([z64, z64, p['r3_W']], axis=1)], axis=0)
    bcm = jnp.concatenate([p['r1_b'], p['r2_b'], p['r3_b']])
    Xm = jnp.concatenate([fv1, feature_2_states, feature_3_states], axis=1)

    z200 = jnp.zeros((64, 200), jnp.float32)
    Rs = jnp.concatenate([
        jnp.concatenate([p['r4_W'], z64, z64], axis=1),
        jnp.concatenate([z200, p['r5_W'], z64], axis=1),
        jnp.concatenate([z200, z64, p['r6_W']], axis=1)], axis=0)
    bcs = jnp.concatenate([p['r4_b'], p['r5_b'], p['r6_b']])
    Xs = jnp.concatenate([feature_4, feature_5_states, fv6], axis=1)

    zeros128 = jnp.zeros((_NPAD, 128), jnp.float32)
    ones128 = jnp.ones((_CH, 128), jnp.float32)

    m_out = _graph_side(Xm, Rm, bcm, edge_index_1,
                        p['convm_W0'], p['convm_b0'],
                        p['convm_W1'], p['convm_b1'],
                        p['r7_W'], p['r7_b'], zeros128, ones128)
    s_out = _graph_side(Xs, Rs, bcs, edge_index_2,
                        p['convs_W0'], p['convs_b0'],
                        p['convs_W1'], p['convs_b1'],
                        p['r8_W'], p['r8_b'], zeros128, ones128)
    return (m_out, s_out)


# speedup vs baseline: 6.6479x; 1.3478x over previous
import functools

import jax
import jax.numpy as jnp
from jax import lax
from jax.experimental import pallas as pl
from jax.experimental.pallas import tpu as pltpu
from jax.experimental.pallas import tpu_sc as plsc

_N = 10000
_E = 320000
_T = 20
_D = 128
_H3 = 192
_NC = 2
_NS = 16
_NPAD = 10112
_RPT = _NPAD // _NS
_CH = 128
_EPAD = 327680
_EPT_DEG = _EPAD // (_NC * _NS)
_EPT_SC = _EPAD // _NS

_BG = 1000
_BL = 1000


def _gru_body(x_ref, wr, wz, wn, ur, uz, un, br, bz, bn, bhn, out_ref):
    B = x_ref.shape[0]
    h = jnp.zeros((B, _H3), jnp.float32)
    for t in range(_T):
        xt = x_ref[:, t, :]
        r = jax.nn.sigmoid(
            jnp.dot(xt, wr[...], preferred_element_type=jnp.float32)
            + jnp.dot(h, ur[...], preferred_element_type=jnp.float32) + br[...])
        z = jax.nn.sigmoid(
            jnp.dot(xt, wz[...], preferred_element_type=jnp.float32)
            + jnp.dot(h, uz[...], preferred_element_type=jnp.float32) + bz[...])
        n = jnp.tanh(
            jnp.dot(xt, wn[...], preferred_element_type=jnp.float32) + bn[...]
            + r * (jnp.dot(h, un[...], preferred_element_type=jnp.float32) + bhn[...]))
        h = (1.0 - z) * n + z * h
    out_ref[...] = h


def _gru(x, wr, wz, wn, ur, uz, un, br, bz, bn, bhn):
    grid = _N // _BG
    wspec = pl.BlockSpec((_D, _H3), lambda i: (0, 0))
    uspec = pl.BlockSpec((_H3, _H3), lambda i: (0, 0))
    bspec = pl.BlockSpec((1, _H3), lambda i: (0, 0))
    return pl.pallas_call(
        _gru_body,
        grid=(grid,),
        in_specs=[pl.BlockSpec((_BG, _T, _D), lambda i: (i, 0, 0)),
                  wspec, wspec, wspec, uspec, uspec, uspec,
                  bspec, bspec, bspec, bspec],
        out_specs=pl.BlockSpec((_BG, _H3), lambda i: (i, 0)),
        out_shape=jax.ShapeDtypeStruct((_N, _H3), jnp.float32),
    )(x, wr, wz, wn, ur, uz, un, br, bz, bn, bhn)


def _dinv(dga_ref, dgb_ref):
    deg = dga_ref[:, 0:1] + dgb_ref[:, 0:1] + 1.0
    return lax.rsqrt(deg)


def _split_out(hs, outa, outb):
    B = hs.shape[0]
    outa[0] = hs[:, :128]
    outb[0] = jnp.concatenate(
        [hs[:, 128:], jnp.zeros((B, 64), jnp.float32)], axis=1)


def _join(a_ref, b_ref):
    return jnp.concatenate([a_ref[0], b_ref[0][:, :64]], axis=1)


def _layer0_body(x_ref, a_ref, c_ref, dga_ref, dgb_ref, outa, outb):
    di = _dinv(dga_ref, dgb_ref)
    h = jnp.dot(x_ref[...], a_ref[...],
                preferred_element_type=jnp.float32) + c_ref[...]
    _split_out(h * di, outa, outb)


def _hs_specs():
    return [pl.BlockSpec((1, _BL, 128), lambda i: (0, i, 0)),
            pl.BlockSpec((1, _BL, 128), lambda i: (0, i, 0))]


def _hs_shapes():
    return [jax.ShapeDtypeStruct((1, _N, 128), jnp.float32),
            jax.ShapeDtypeStruct((1, _N, 128), jnp.float32)]


def _layer0(x, a, c, dga, dgb):
    K = x.shape[1]
    grid = _N // _BL
    outa, outb = pl.pallas_call(
        _layer0_body,
        grid=(grid,),
        in_specs=[pl.BlockSpec((_BL, K), lambda i: (i, 0)),
                  pl.BlockSpec((K, _H3), lambda i: (0, 0)),
                  pl.BlockSpec((1, _H3), lambda i: (0, 0)),
                  pl.BlockSpec((_BL, 128), lambda i: (i, 0)),
                  pl.BlockSpec((_BL, 128), lambda i: (i, 0))],
        out_specs=_hs_specs(),
        out_shape=_hs_shapes(),
    )(x, a, c, dga, dgb)
    return jnp.concatenate([outa, outb], axis=0)


def _layer1_body(aca_ref, acb_ref, hsa_ref, hsb_ref, dga_ref, dgb_ref,
                 b_ref, w_ref, outa, outb):
    di = _dinv(dga_ref, dgb_ref)
    x = di * (_join(aca_ref, acb_ref) + _join(hsa_ref, hsb_ref)) + b_ref[...]
    x = jnp.maximum(x, 0.0)
    h = jnp.dot(x, w_ref[...], preferred_element_type=jnp.float32)
    _split_out(h * di, outa, outb)


def _acc_specs():
    return [pl.BlockSpec((1, _BL, 128), lambda i: (0, i, 0)),
            pl.BlockSpec((1, _BL, 128), lambda i: (1, i, 0))]


def _layer1(acc, hs, dga, dgb, b, w):
    grid = _N // _BL
    outa, outb = pl.pallas_call(
        _layer1_body,
        grid=(grid,),
        in_specs=_acc_specs() + _acc_specs() + [
            pl.BlockSpec((_BL, 128), lambda i: (i, 0)),
            pl.BlockSpec((_BL, 128), lambda i: (i, 0)),
            pl.BlockSpec((1, _H3), lambda i: (0, 0)),
            pl.BlockSpec((_H3, _H3), lambda i: (0, 0))],
        out_specs=_hs_specs(),
        out_shape=_hs_shapes(),
    )(acc, acc, hs, hs, dga, dgb, b, w)
    return jnp.concatenate([outa, outb], axis=0)


def _final_body(aca_ref, acb_ref, hsa_ref, hsb_ref, dga_ref, dgb_ref,
                b_ref, w2_ref, b2_ref, out_ref):
    di = _dinv(dga_ref, dgb_ref)
    x = di * (_join(aca_ref, acb_ref) + _join(hsa_ref, hsb_ref)) + b_ref[...]
    z = jnp.dot(x, w2_ref[...], preferred_element_type=jnp.float32) + b2_ref[...]
    m = jnp.max(z, axis=1, keepdims=True)
    e = jnp.exp(z - m)
    out_ref[...] = e / jnp.sum(e, axis=1, keepdims=True)


def _final(acc, hs, dga, dgb, b, w2, b2):
    grid = _N // _BL
    return pl.pallas_call(
        _final_body,
        grid=(grid,),
        in_specs=_acc_specs() + _acc_specs() + [
            pl.BlockSpec((_BL, 128), lambda i: (i, 0)),
            pl.BlockSpec((_BL, 128), lambda i: (i, 0)),
            pl.BlockSpec((1, _H3), lambda i: (0, 0)),
            pl.BlockSpec((_H3, 128), lambda i: (0, 0)),
            pl.BlockSpec((1, 128), lambda i: (0, 0))],
        out_specs=pl.BlockSpec((_BL, 128), lambda i: (i, 0)),
        out_shape=jax.ShapeDtypeStruct((_N, 128), jnp.float32),
    )(acc, acc, hs, hs, dga, dgb, b, w2, b2)


_NCHUNK = _EPT_SC // _CH
_NSTG = 4
_CPS = _NCHUNK // _NSTG
_JPS = _CPS // 2
_NCH_DEG = _EPT_DEG // _CH
_ZR = _RPT - 4 * _CH


def _sc_mesh():
    return plsc.VectorSubcoreMesh(core_axis_name="c", subcore_axis_name="s")


def _sc_degree(dst2d, zeros128, ones128):
    @functools.partial(
        pl.kernel,
        out_type=jax.ShapeDtypeStruct((_NC, _NPAD, 128), jnp.float32),
        mesh=_sc_mesh(),
        scratch_types=[
            pltpu.VMEM((_NCH_DEG, _CH), jnp.int32),
            pltpu.VMEM((_CH, 128), jnp.float32),
            pltpu.VMEM_SHARED((_NPAD, 128), jnp.float32),
        ],
    )
    def k(dst_hbm, zeros_hbm, ones_hbm, out_hbm, idx2d, ones_v, acc_sh):
        c = lax.axis_index("c")
        s = lax.axis_index("s")
        r0 = s * _RPT
        pltpu.sync_copy(zeros_hbm.at[pl.ds(r0, _RPT)],
                        acc_sh.at[pl.ds(r0, _RPT)])
        pltpu.sync_copy(ones_hbm, ones_v)
        row0 = (c * _NS + s) * _NCH_DEG
        pltpu.sync_copy(dst_hbm.at[pl.ds(row0, _NCH_DEG)], idx2d)
        plsc.subcore_barrier()

        def body(i, carry):
            pltpu.sync_copy(ones_v, acc_sh.at[idx2d.at[i]], add=True)
            return carry

        lax.fori_loop(0, _NCH_DEG, body, 0)
        plsc.subcore_barrier()
        pltpu.sync_copy(acc_sh.at[pl.ds(r0, _RPT)],
                        out_hbm.at[c].at[pl.ds(r0, _RPT)])

    return k(dst2d, zeros128, ones128)


def _sc_scatter(hs, src2d, dst2d, zeros128):
    @functools.partial(
        pl.kernel,
        out_type=jax.ShapeDtypeStruct((_NC, _NPAD, 128), jnp.float32),
        mesh=_sc_mesh(),
        scratch_types=[
            pltpu.VMEM((_CPS, _CH), jnp.int32),
            pltpu.VMEM((_CPS, _CH), jnp.int32),
            pltpu.VMEM((_CH, 128), jnp.float32),
            pltpu.VMEM((_CH, 128), jnp.float32),
            pltpu.VMEM_SHARED((_NPAD, 128), jnp.float32),
            pltpu.SemaphoreType.DMA,
            pltpu.SemaphoreType.DMA,
        ],
    )
    def k(hs_hbm, src_hbm, dst_hbm, zeros_hbm, out_hbm,
          src_st, dst_st, rows0, rows1, acc_sh, sem0, sem1):
        c = lax.axis_index("c")
        s = lax.axis_index("s")
        r0 = s * _RPT
        pltpu.sync_copy(zeros_hbm.at[pl.ds(r0, _RPT)],
                        acc_sh.at[pl.ds(r0, _RPT)])
        plsc.subcore_barrier()
        row0 = s * _NCHUNK
        half = hs_hbm.at[c]

        for stg in range(_NSTG):
            pltpu.sync_copy(src_hbm.at[pl.ds(row0 + stg * _CPS, _CPS)],
                            src_st)
            pltpu.sync_copy(dst_hbm.at[pl.ds(row0 + stg * _CPS, _CPS)],
                            dst_st)
            pltpu.async_copy(half.at[src_st.at[0]], rows0, sem0)

            def body(j, carry):
                i0 = j * 2
                pltpu.async_copy(half.at[src_st.at[i0 + 1]], rows1, sem1)
                pltpu.make_async_copy(
                    half.at[pl.ds(0, _CH)], rows0, sem0).wait()
                pltpu.sync_copy(rows0, acc_sh.at[dst_st.at[i0]], add=True)

                @pl.when(j < _JPS - 1)
                def _():
                    pltpu.async_copy(half.at[src_st.at[i0 + 2]], rows0, sem0)

                pltpu.make_async_copy(
                    half.at[pl.ds(0, _CH)], rows1, sem1).wait()
                pltpu.sync_copy(rows1, acc_sh.at[dst_st.at[i0 + 1]],
                                add=True)
                return carry

            lax.fori_loop(0, _JPS, body, 0)

        plsc.subcore_barrier()
        pltpu.sync_copy(acc_sh.at[pl.ds(r0, _RPT)],
                        out_hbm.at[c].at[pl.ds(r0, _RPT)])

    return k(hs, src2d, dst2d, zeros128)


def _gru_prep(Wi, Wh, bi, bh):
    WiT = Wi.T
    WhT = Wh.T
    wr, wz, wn = WiT[:, :_H3], WiT[:, _H3:2 * _H3], WiT[:, 2 * _H3:]
    ur, uz, un = WhT[:, :_H3], WhT[:, _H3:2 * _H3], WhT[:, 2 * _H3:]
    br = (bi[:_H3] + bh[:_H3])[None, :]
    bz = (bi[_H3:2 * _H3] + bh[_H3:2 * _H3])[None, :]
    bn = bi[2 * _H3:][None, :]
    bhn = bh[2 * _H3:][None, :]
    return wr, wz, wn, ur, uz, un, br, bz, bn, bhn


def _pad_edges(ei):
    src, dst = ei[0], ei[1]
    npad = _EPAD - _E
    src_pad = jnp.concatenate([src, jnp.zeros((npad,), jnp.int32)])
    dst_pad = jnp.concatenate([dst, jnp.full((npad,), _N, jnp.int32)])
    return (src_pad.reshape(_EPAD // _CH, _CH),
            dst_pad.reshape(_EPAD // _CH, _CH))


def _graph_side(xcat, R, bcat, ei, W0, b0, W1, b1, Wf, bf,
                zeros128, ones128):
    src2d, dst2d = _pad_edges(ei)
    degp = _sc_degree(dst2d, zeros128, ones128)
    dga, dgb = degp[0, :_N], degp[1, :_N]

    A = jnp.dot(R.T, W0.T)
    c = jnp.dot(bcat, W0.T)[None, :]
    hs0 = _layer0(xcat, A, c, dga, dgb)
    acc0 = _sc_scatter(hs0, src2d, dst2d, zeros128)[:, :_N]
    hs1 = _layer1(acc0, hs0, dga, dgb, b0[None, :], W1.T)
    acc1 = _sc_scatter(hs1, src2d, dst2d, zeros128)[:, :_N]

    w2 = jnp.zeros((_H3, 128), jnp.float32).at[:, :2].set(Wf.T)
    b2 = jnp.full((128,), -1e30, jnp.float32).at[:2].set(bf)[None, :]
    out = _final(acc1, hs1, dga, dgb, b1[None, :], w2, b2)
    return out[:, :2]


def kernel(feature_1, feature_2_states, feature_3_states, feature_4,
           feature_5_states, feature_6, edge_index_1, edge_index_2, params):
    p = params
    fv1 = _gru(feature_1, *_gru_prep(p['gru1_Wi'], p['gru1_Wh'],
                                     p['gru1_bi'], p['gru1_bh']))
    fv6 = _gru(feature_6, *_gru_prep(p['gru2_Wi'], p['gru2_Wh'],
                                     p['gru2_bi'], p['gru2_bh']))

    z64 = jnp.zeros((64, _H3), jnp.float32)
    Rm = jnp.concatenate([
        jnp.concatenate([p['r1_W'], z64, z64], axis=1),
        jnp.concatenate([z64, p['r2_W'], z64], axis=1),
        jnp.concatenate([z64, z64, p['r3_W']], axis=1)], axis=0)
    bcm = jnp.concatenate([p['r1_b'], p['r2_b'], p['r3_b']])
    Xm = jnp.concatenate([fv1, feature_2_states, feature_3_states], axis=1)

    z200 = jnp.zeros((64, 200), jnp.float32)
    Rs = jnp.concatenate([
        jnp.concatenate([p['r4_W'], z64, z64], axis=1),
        jnp.concatenate([z200, p['r5_W'], z64], axis=1),
        jnp.concatenate([z200, z64, p['r6_W']], axis=1)], axis=0)
    bcs = jnp.concatenate([p['r4_b'], p['r5_b'], p['r6_b']])
    Xs = jnp.concatenate([feature_4, feature_5_states, fv6], axis=1)

    zeros128 = jnp.zeros((_NPAD, 128), jnp.float32)
    ones128 = jnp.ones((_CH, 128), jnp.float32)

    m_out = _graph_side(Xm, Rm, bcm, edge_index_1,
                        p['convm_W0'], p['convm_b0'],
                        p['convm_W1'], p['convm_b1'],
                        p['r7_W'], p['r7_b'], zeros128, ones128)
    s_out = _graph_side(Xs, Rs, bcs, edge_index_2,
                        p['convs_W0'], p['convs_b0'],
                        p['convs_W1'], p['convs_b1'],
                        p['r8_W'], p['r8_b'], zeros128, ones128)
    return (m_out, s_out)


# speedup vs baseline: 7.1199x; 1.0710x over previous
import functools

import jax
import jax.numpy as jnp
from jax import lax
from jax.experimental import pallas as pl
from jax.experimental.pallas import tpu as pltpu
from jax.experimental.pallas import tpu_sc as plsc

_N = 10000
_E = 320000
_T = 20
_D = 128
_H3 = 192
_NC = 2
_NS = 16
_NPAD = 10112
_RPT = _NPAD // _NS
_CH = 128
_EPAD = 327680
_EPT_DEG = _EPAD // (_NC * _NS)
_EPT_SC = _EPAD // _NS

_BG = 1000
_BL = 1000


def _gru_body(x_ref, wr, wz, wn, ur, uz, un, br, bz, bn, bhn, out_ref):
    B = x_ref.shape[0]
    h = jnp.zeros((B, _H3), jnp.float32)
    for t in range(_T):
        xt = x_ref[:, t, :]
        r = jax.nn.sigmoid(
            jnp.dot(xt, wr[...], preferred_element_type=jnp.float32)
            + jnp.dot(h, ur[...], preferred_element_type=jnp.float32) + br[...])
        z = jax.nn.sigmoid(
            jnp.dot(xt, wz[...], preferred_element_type=jnp.float32)
            + jnp.dot(h, uz[...], preferred_element_type=jnp.float32) + bz[...])
        n = jnp.tanh(
            jnp.dot(xt, wn[...], preferred_element_type=jnp.float32) + bn[...]
            + r * (jnp.dot(h, un[...], preferred_element_type=jnp.float32) + bhn[...]))
        h = (1.0 - z) * n + z * h
    out_ref[...] = h


def _gru(x, wr, wz, wn, ur, uz, un, br, bz, bn, bhn):
    grid = _N // _BG
    wspec = pl.BlockSpec((_D, _H3), lambda i: (0, 0))
    uspec = pl.BlockSpec((_H3, _H3), lambda i: (0, 0))
    bspec = pl.BlockSpec((1, _H3), lambda i: (0, 0))
    return pl.pallas_call(
        _gru_body,
        grid=(grid,),
        in_specs=[pl.BlockSpec((_BG, _T, _D), lambda i: (i, 0, 0)),
                  wspec, wspec, wspec, uspec, uspec, uspec,
                  bspec, bspec, bspec, bspec],
        out_specs=pl.BlockSpec((_BG, _H3), lambda i: (i, 0)),
        out_shape=jax.ShapeDtypeStruct((_N, _H3), jnp.float32),
    )(x, wr, wz, wn, ur, uz, un, br, bz, bn, bhn)


def _dinv(dga_ref, dgb_ref):
    deg = dga_ref[:, 0:1] + dgb_ref[:, 0:1] + 1.0
    return lax.rsqrt(deg)


def _split_out(hs, outa, outb):
    B = hs.shape[0]
    outa[0] = hs[:, :128]
    outb[0] = jnp.concatenate(
        [hs[:, 128:], jnp.zeros((B, 64), jnp.float32)], axis=1)


def _join(a_ref, b_ref):
    return jnp.concatenate([a_ref[0], b_ref[0][:, :64]], axis=1)


def _layer0_body(x_ref, a_ref, c_ref, dga_ref, dgb_ref, outa, outb):
    di = _dinv(dga_ref, dgb_ref)
    h = jnp.dot(x_ref[...], a_ref[...],
                preferred_element_type=jnp.float32) + c_ref[...]
    _split_out(h * di, outa, outb)


def _hs_specs():
    return [pl.BlockSpec((1, _BL, 128), lambda i: (0, i, 0)),
            pl.BlockSpec((1, _BL, 128), lambda i: (0, i, 0))]


def _hs_shapes():
    return [jax.ShapeDtypeStruct((1, _N, 128), jnp.float32),
            jax.ShapeDtypeStruct((1, _N, 128), jnp.float32)]


def _layer0(x, a, c, dga, dgb):
    K = x.shape[1]
    grid = _N // _BL
    outa, outb = pl.pallas_call(
        _layer0_body,
        grid=(grid,),
        in_specs=[pl.BlockSpec((_BL, K), lambda i: (i, 0)),
                  pl.BlockSpec((K, _H3), lambda i: (0, 0)),
                  pl.BlockSpec((1, _H3), lambda i: (0, 0)),
                  pl.BlockSpec((_BL, 128), lambda i: (i, 0)),
                  pl.BlockSpec((_BL, 128), lambda i: (i, 0))],
        out_specs=_hs_specs(),
        out_shape=_hs_shapes(),
    )(x, a, c, dga, dgb)
    return jnp.concatenate([outa, outb], axis=0)


def _layer1_body(aca_ref, acb_ref, hsa_ref, hsb_ref, dga_ref, dgb_ref,
                 b_ref, w_ref, outa, outb):
    di = _dinv(dga_ref, dgb_ref)
    x = di * (_join(aca_ref, acb_ref) + _join(hsa_ref, hsb_ref)) + b_ref[...]
    x = jnp.maximum(x, 0.0)
    h = jnp.dot(x, w_ref[...], preferred_element_type=jnp.float32)
    _split_out(h * di, outa, outb)


def _acc_specs():
    return [pl.BlockSpec((1, _BL, 128), lambda i: (0, i, 0)),
            pl.BlockSpec((1, _BL, 128), lambda i: (1, i, 0))]


def _layer1(acc, hs, dga, dgb, b, w):
    grid = _N // _BL
    outa, outb = pl.pallas_call(
        _layer1_body,
        grid=(grid,),
        in_specs=_acc_specs() + _acc_specs() + [
            pl.BlockSpec((_BL, 128), lambda i: (i, 0)),
            pl.BlockSpec((_BL, 128), lambda i: (i, 0)),
            pl.BlockSpec((1, _H3), lambda i: (0, 0)),
            pl.BlockSpec((_H3, _H3), lambda i: (0, 0))],
        out_specs=_hs_specs(),
        out_shape=_hs_shapes(),
    )(acc, acc, hs, hs, dga, dgb, b, w)
    return jnp.concatenate([outa, outb], axis=0)


def _proj_body(hsa_ref, hsb_ref, w2_ref, out_ref):
    out_ref[...] = jnp.dot(_join(hsa_ref, hsb_ref), w2_ref[...],
                           preferred_element_type=jnp.float32)


def _proj(hs, w2):
    grid = _N // _BL
    return pl.pallas_call(
        _proj_body,
        grid=(grid,),
        in_specs=_acc_specs() + [pl.BlockSpec((_H3, 128), lambda i: (0, 0))],
        out_specs=pl.BlockSpec((_BL, 128), lambda i: (i, 0)),
        out_shape=jax.ShapeDtypeStruct((_N, 128), jnp.float32),
    )(hs, hs, w2)


def _final_body(sya_ref, syb_ref, hsa_ref, hsb_ref, dga_ref, dgb_ref,
                w2_ref, cb_ref, out_ref):
    di = _dinv(dga_ref, dgb_ref)
    z = (jnp.dot(di * _join(hsa_ref, hsb_ref), w2_ref[...],
                 preferred_element_type=jnp.float32)
         + di * (sya_ref[0] + syb_ref[0]) + cb_ref[...])
    m = jnp.max(z, axis=1, keepdims=True)
    e = jnp.exp(z - m)
    out_ref[...] = e / jnp.sum(e, axis=1, keepdims=True)


def _final(sy, hs, dga, dgb, w2, cb):
    grid = _N // _BL
    return pl.pallas_call(
        _final_body,
        grid=(grid,),
        in_specs=_acc_specs() + _acc_specs() + [
            pl.BlockSpec((_BL, 128), lambda i: (i, 0)),
            pl.BlockSpec((_BL, 128), lambda i: (i, 0)),
            pl.BlockSpec((_H3, 128), lambda i: (0, 0)),
            pl.BlockSpec((1, 128), lambda i: (0, 0))],
        out_specs=pl.BlockSpec((_BL, 128), lambda i: (i, 0)),
        out_shape=jax.ShapeDtypeStruct((_N, 128), jnp.float32),
    )(sy, sy, hs, hs, dga, dgb, w2, cb)


_NCHUNK = _EPT_SC // _CH
_NSTG = 4
_CPS = _NCHUNK // _NSTG
_JPS = _CPS // 2
_NCH_DEG = _EPT_DEG // _CH
_ZR = _RPT - 4 * _CH


def _sc_mesh():
    return plsc.VectorSubcoreMesh(core_axis_name="c", subcore_axis_name="s")


def _sc_degree(dst2d, zeros128, ones128):
    @functools.partial(
        pl.kernel,
        out_type=jax.ShapeDtypeStruct((_NC, _NPAD, 128), jnp.float32),
        mesh=_sc_mesh(),
        scratch_types=[
            pltpu.VMEM((_NCH_DEG, _CH), jnp.int32),
            pltpu.VMEM((_CH, 128), jnp.float32),
            pltpu.VMEM_SHARED((_NPAD, 128), jnp.float32),
        ],
    )
    def k(dst_hbm, zeros_hbm, ones_hbm, out_hbm, idx2d, ones_v, acc_sh):
        c = lax.axis_index("c")
        s = lax.axis_index("s")
        r0 = s * _RPT
        pltpu.sync_copy(zeros_hbm.at[pl.ds(r0, _RPT)],
                        acc_sh.at[pl.ds(r0, _RPT)])
        pltpu.sync_copy(ones_hbm, ones_v)
        row0 = (c * _NS + s) * _NCH_DEG
        pltpu.sync_copy(dst_hbm.at[pl.ds(row0, _NCH_DEG)], idx2d)
        plsc.subcore_barrier()

        def body(i, carry):
            pltpu.sync_copy(ones_v, acc_sh.at[idx2d.at[i]], add=True)
            return carry

        lax.fori_loop(0, _NCH_DEG, body, 0)
        plsc.subcore_barrier()
        pltpu.sync_copy(acc_sh.at[pl.ds(r0, _RPT)],
                        out_hbm.at[c].at[pl.ds(r0, _RPT)])

    return k(dst2d, zeros128, ones128)


def _sc_scatter(hs, src2d, dst2d, zeros128):
    @functools.partial(
        pl.kernel,
        out_type=jax.ShapeDtypeStruct((_NC, _NPAD, 128), jnp.float32),
        mesh=_sc_mesh(),
        scratch_types=[
            pltpu.VMEM((_CPS, _CH), jnp.int32),
            pltpu.VMEM((_CPS, _CH), jnp.int32),
            pltpu.VMEM((_CH, 128), jnp.float32),
            pltpu.VMEM((_CH, 128), jnp.float32),
            pltpu.VMEM_SHARED((_NPAD, 128), jnp.float32),
            pltpu.SemaphoreType.DMA,
            pltpu.SemaphoreType.DMA,
        ],
    )
    def k(hs_hbm, src_hbm, dst_hbm, zeros_hbm, out_hbm,
          src_st, dst_st, rows0, rows1, acc_sh, sem0, sem1):
        c = lax.axis_index("c")
        s = lax.axis_index("s")
        r0 = s * _RPT
        pltpu.sync_copy(zeros_hbm.at[pl.ds(r0, _RPT)],
                        acc_sh.at[pl.ds(r0, _RPT)])
        plsc.subcore_barrier()
        row0 = s * _NCHUNK
        half = hs_hbm.at[c]

        for stg in range(_NSTG):
            pltpu.sync_copy(src_hbm.at[pl.ds(row0 + stg * _CPS, _CPS)],
                            src_st)
            pltpu.sync_copy(dst_hbm.at[pl.ds(row0 + stg * _CPS, _CPS)],
                            dst_st)
            pltpu.async_copy(half.at[src_st.at[0]], rows0, sem0)

            def body(j, carry):
                i0 = j * 2
                pltpu.async_copy(half.at[src_st.at[i0 + 1]], rows1, sem1)
                pltpu.make_async_copy(
                    half.at[pl.ds(0, _CH)], rows0, sem0).wait()
                pltpu.sync_copy(rows0, acc_sh.at[dst_st.at[i0]], add=True)

                @pl.when(j < _JPS - 1)
                def _():
                    pltpu.async_copy(half.at[src_st.at[i0 + 2]], rows0, sem0)

                pltpu.make_async_copy(
                    half.at[pl.ds(0, _CH)], rows1, sem1).wait()
                pltpu.sync_copy(rows1, acc_sh.at[dst_st.at[i0 + 1]],
                                add=True)
                return carry

            lax.fori_loop(0, _JPS, body, 0)

        plsc.subcore_barrier()
        pltpu.sync_copy(acc_sh.at[pl.ds(r0, _RPT)],
                        out_hbm.at[c].at[pl.ds(r0, _RPT)])

    return k(hs, src2d, dst2d, zeros128)


_CPS2 = _NCH_DEG // 2
_JPS2 = _CPS2 // 2


def _sc_scatter_thin(y, src2d, dst2d, zeros128):
    @functools.partial(
        pl.kernel,
        out_type=jax.ShapeDtypeStruct((_NC, _NPAD, 128), jnp.float32),
        mesh=_sc_mesh(),
        scratch_types=[
            pltpu.VMEM((_CPS2, _CH), jnp.int32),
            pltpu.VMEM((_CPS2, _CH), jnp.int32),
            pltpu.VMEM((_CH, 128), jnp.float32),
            pltpu.VMEM((_CH, 128), jnp.float32),
            pltpu.VMEM_SHARED((_NPAD, 128), jnp.float32),
            pltpu.SemaphoreType.DMA,
            pltpu.SemaphoreType.DMA,
        ],
    )
    def k(y_hbm, src_hbm, dst_hbm, zeros_hbm, out_hbm,
          src_st, dst_st, rows0, rows1, acc_sh, sem0, sem1):
        c = lax.axis_index("c")
        s = lax.axis_index("s")
        r0 = s * _RPT
        pltpu.sync_copy(zeros_hbm.at[pl.ds(r0, _RPT)],
                        acc_sh.at[pl.ds(r0, _RPT)])
        plsc.subcore_barrier()
        base = (c * _NS + s) * _NCH_DEG

        for stg in range(2):
            pltpu.sync_copy(src_hbm.at[pl.ds(base + stg * _CPS2, _CPS2)],
                            src_st)
            pltpu.sync_copy(dst_hbm.at[pl.ds(base + stg * _CPS2, _CPS2)],
                            dst_st)
            pltpu.async_copy(y_hbm.at[src_st.at[0]], rows0, sem0)

            def body(j, carry):
                i0 = j * 2
                pltpu.async_copy(y_hbm.at[src_st.at[i0 + 1]], rows1, sem1)
                pltpu.make_async_copy(
                    y_hbm.at[pl.ds(0, _CH)], rows0, sem0).wait()
                pltpu.sync_copy(rows0, acc_sh.at[dst_st.at[i0]], add=True)

                @pl.when(j < _JPS2 - 1)
                def _():
                    pltpu.async_copy(y_hbm.at[src_st.at[i0 + 2]], rows0,
                                     sem0)

                pltpu.make_async_copy(
                    y_hbm.at[pl.ds(0, _CH)], rows1, sem1).wait()
                pltpu.sync_copy(rows1, acc_sh.at[dst_st.at[i0 + 1]],
                                add=True)
                return carry

            lax.fori_loop(0, _JPS2, body, 0)

        plsc.subcore_barrier()
        pltpu.sync_copy(acc_sh.at[pl.ds(r0, _RPT)],
                        out_hbm.at[c].at[pl.ds(r0, _RPT)])

    return k(y, src2d, dst2d, zeros128)


def _gru_prep(Wi, Wh, bi, bh):
    WiT = Wi.T
    WhT = Wh.T
    wr, wz, wn = WiT[:, :_H3], WiT[:, _H3:2 * _H3], WiT[:, 2 * _H3:]
    ur, uz, un = WhT[:, :_H3], WhT[:, _H3:2 * _H3], WhT[:, 2 * _H3:]
    br = (bi[:_H3] + bh[:_H3])[None, :]
    bz = (bi[_H3:2 * _H3] + bh[_H3:2 * _H3])[None, :]
    bn = bi[2 * _H3:][None, :]
    bhn = bh[2 * _H3:][None, :]
    return wr, wz, wn, ur, uz, un, br, bz, bn, bhn


def _pad_edges(ei):
    src, dst = ei[0], ei[1]
    npad = _EPAD - _E
    src_pad = jnp.concatenate([src, jnp.zeros((npad,), jnp.int32)])
    dst_pad = jnp.concatenate([dst, jnp.full((npad,), _N, jnp.int32)])
    return (src_pad.reshape(_EPAD // _CH, _CH),
            dst_pad.reshape(_EPAD // _CH, _CH))


def _graph_side(xcat, R, bcat, ei, W0, b0, W1, b1, Wf, bf,
                zeros128, ones128):
    src2d, dst2d = _pad_edges(ei)
    degp = _sc_degree(dst2d, zeros128, ones128)
    dga, dgb = degp[0, :_N], degp[1, :_N]

    A = jnp.dot(R.T, W0.T)
    c = jnp.dot(bcat, W0.T)[None, :]
    hs0 = _layer0(xcat, A, c, dga, dgb)
    acc0 = _sc_scatter(hs0, src2d, dst2d, zeros128)[:, :_N]
    hs1 = _layer1(acc0, hs0, dga, dgb, b0[None, :], W1.T)

    w2 = jnp.zeros((_H3, 128), jnp.float32).at[:, :2].set(Wf.T)
    b2 = jnp.full((128,), -1e30, jnp.float32).at[:2].set(bf)
    cb = (jnp.dot(b1, w2) + b2)[None, :]
    y = _proj(hs1, w2)
    sy = _sc_scatter_thin(y, src2d, dst2d, zeros128)[:, :_N]
    out = _final(sy, hs1, dga, dgb, w2, cb)
    return out[:, :2]


def kernel(feature_1, feature_2_states, feature_3_states, feature_4,
           feature_5_states, feature_6, edge_index_1, edge_index_2, params):
    p = params
    fv1 = _gru(feature_1, *_gru_prep(p['gru1_Wi'], p['gru1_Wh'],
                                     p['gru1_bi'], p['gru1_bh']))
    fv6 = _gru(feature_6, *_gru_prep(p['gru2_Wi'], p['gru2_Wh'],
                                     p['gru2_bi'], p['gru2_bh']))

    z64 = jnp.zeros((64, _H3), jnp.float32)
    Rm = jnp.concatenate([
        jnp.concatenate([p['r1_W'], z64, z64], axis=1),
        jnp.concatenate([z64, p['r2_W'], z64], axis=1),
        jnp.concatenate([z64, z64, p['r3_W']], axis=1)], axis=0)
    bcm = jnp.concatenate([p['r1_b'], p['r2_b'], p['r3_b']])
    Xm = jnp.concatenate([fv1, feature_2_states, feature_3_states], axis=1)

    z200 = jnp.zeros((64, 200), jnp.float32)
    Rs = jnp.concatenate([
        jnp.concatenate([p['r4_W'], z64, z64], axis=1),
        jnp.concatenate([z200, p['r5_W'], z64], axis=1),
        jnp.concatenate([z200, z64, p['r6_W']], axis=1)], axis=0)
    bcs = jnp.concatenate([p['r4_b'], p['r5_b'], p['r6_b']])
    Xs = jnp.concatenate([feature_4, feature_5_states, fv6], axis=1)

    zeros128 = jnp.zeros((_NPAD, 128), jnp.float32)
    ones128 = jnp.ones((_CH, 128), jnp.float32)

    m_out = _graph_side(Xm, Rm, bcm, edge_index_1,
                        p['convm_W0'], p['convm_b0'],
                        p['convm_W1'], p['convm_b1'],
                        p['r7_W'], p['r7_b'], zeros128, ones128)
    s_out = _graph_side(Xs, Rs, bcs, edge_index_2,
                        p['convs_W0'], p['convs_b0'],
                        p['convs_W1'], p['convs_b1'],
                        p['r8_W'], p['r8_b'], zeros128, ones128)
    return (m_out, s_out)


# speedup vs baseline: 7.4079x; 1.0404x over previous
import functools

import jax
import jax.numpy as jnp
from jax import lax
from jax.experimental import pallas as pl
from jax.experimental.pallas import tpu as pltpu
from jax.experimental.pallas import tpu_sc as plsc

_N = 10000
_E = 320000
_T = 20
_D = 128
_H3 = 192
_NC = 2
_NS = 16
_NPAD = 10112
_RPT = _NPAD // _NS
_CH = 128
_EPAD = 327680
_EPT_DEG = _EPAD // (_NC * _NS)
_EPT_SC = _EPAD // _NS

_BG = 1000
_BL = 1000


def _gru_body(x_ref, wr, wz, wn, ur, uz, un, br, bz, bn, bhn, out_ref):
    B = x_ref.shape[0]
    h = jnp.zeros((B, _H3), jnp.float32)
    for t in range(_T):
        xt = x_ref[:, t, :]
        r = jax.nn.sigmoid(
            jnp.dot(xt, wr[...], preferred_element_type=jnp.float32)
            + jnp.dot(h, ur[...], preferred_element_type=jnp.float32) + br[...])
        z = jax.nn.sigmoid(
            jnp.dot(xt, wz[...], preferred_element_type=jnp.float32)
            + jnp.dot(h, uz[...], preferred_element_type=jnp.float32) + bz[...])
        n = jnp.tanh(
            jnp.dot(xt, wn[...], preferred_element_type=jnp.float32) + bn[...]
            + r * (jnp.dot(h, un[...], preferred_element_type=jnp.float32) + bhn[...]))
        h = (1.0 - z) * n + z * h
    out_ref[...] = h


def _gru(x, wr, wz, wn, ur, uz, un, br, bz, bn, bhn):
    grid = _N // _BG
    wspec = pl.BlockSpec((_D, _H3), lambda i: (0, 0))
    uspec = pl.BlockSpec((_H3, _H3), lambda i: (0, 0))
    bspec = pl.BlockSpec((1, _H3), lambda i: (0, 0))
    return pl.pallas_call(
        _gru_body,
        grid=(grid,),
        in_specs=[pl.BlockSpec((_BG, _T, _D), lambda i: (i, 0, 0)),
                  wspec, wspec, wspec, uspec, uspec, uspec,
                  bspec, bspec, bspec, bspec],
        out_specs=pl.BlockSpec((_BG, _H3), lambda i: (i, 0)),
        out_shape=jax.ShapeDtypeStruct((_N, _H3), jnp.float32),
    )(x, wr, wz, wn, ur, uz, un, br, bz, bn, bhn)


def _dinv(dga_ref, dgb_ref):
    deg = dga_ref[:, 0:1] + dgb_ref[:, 0:1] + 1.0
    return lax.rsqrt(deg)


def _split_out(hs, outa, outb):
    B = hs.shape[0]
    outa[0] = hs[:, :128]
    outb[0] = jnp.concatenate(
        [hs[:, 128:], jnp.zeros((B, 64), jnp.float32)], axis=1)


def _join(a_ref, b_ref):
    return jnp.concatenate([a_ref[0], b_ref[0][:, :64]], axis=1)


def _layer0_body(x_ref, a_ref, c_ref, dga_ref, dgb_ref, outa, outb):
    di = _dinv(dga_ref, dgb_ref)
    h = jnp.dot(x_ref[...], a_ref[...],
                preferred_element_type=jnp.float32) + c_ref[...]
    _split_out(h * di, outa, outb)


def _hs_specs():
    return [pl.BlockSpec((1, _BL, 128), lambda i: (0, i, 0)),
            pl.BlockSpec((1, _BL, 128), lambda i: (0, i, 0))]


def _hs_shapes():
    return [jax.ShapeDtypeStruct((1, _N, 128), jnp.float32),
            jax.ShapeDtypeStruct((1, _N, 128), jnp.float32)]


def _layer0(x, a, c, dga, dgb):
    K = x.shape[1]
    grid = _N // _BL
    outa, outb = pl.pallas_call(
        _layer0_body,
        grid=(grid,),
        in_specs=[pl.BlockSpec((_BL, K), lambda i: (i, 0)),
                  pl.BlockSpec((K, _H3), lambda i: (0, 0)),
                  pl.BlockSpec((1, _H3), lambda i: (0, 0)),
                  pl.BlockSpec((_BL, 128), lambda i: (i, 0)),
                  pl.BlockSpec((_BL, 128), lambda i: (i, 0))],
        out_specs=_hs_specs(),
        out_shape=_hs_shapes(),
    )(x, a, c, dga, dgb)
    return jnp.concatenate([outa, outb], axis=0)


def _layer1_body(aca_ref, acb_ref, hsa_ref, hsb_ref, dga_ref, dgb_ref,
                 b_ref, w_ref, outa, outb):
    di = _dinv(dga_ref, dgb_ref)
    x = di * (_join(aca_ref, acb_ref) + _join(hsa_ref, hsb_ref)) + b_ref[...]
    x = jnp.maximum(x, 0.0)
    h = jnp.dot(x, w_ref[...], preferred_element_type=jnp.float32)
    _split_out(h * di, outa, outb)


def _acc_specs():
    return [pl.BlockSpec((1, _BL, 128), lambda i: (0, i, 0)),
            pl.BlockSpec((1, _BL, 128), lambda i: (1, i, 0))]


def _layer1(acc, hs, dga, dgb, b, w):
    grid = _N // _BL
    outa, outb = pl.pallas_call(
        _layer1_body,
        grid=(grid,),
        in_specs=_acc_specs() + _acc_specs() + [
            pl.BlockSpec((_BL, 128), lambda i: (i, 0)),
            pl.BlockSpec((_BL, 128), lambda i: (i, 0)),
            pl.BlockSpec((1, _H3), lambda i: (0, 0)),
            pl.BlockSpec((_H3, _H3), lambda i: (0, 0))],
        out_specs=_hs_specs(),
        out_shape=_hs_shapes(),
    )(acc, acc, hs, hs, dga, dgb, b, w)
    return jnp.concatenate([outa, outb], axis=0)


def _proj_body(hsa_ref, hsb_ref, w2_ref, out_ref):
    out_ref[...] = jnp.dot(_join(hsa_ref, hsb_ref), w2_ref[...],
                           preferred_element_type=jnp.float32)


def _proj(hs, w2):
    grid = _N // _BL
    return pl.pallas_call(
        _proj_body,
        grid=(grid,),
        in_specs=_acc_specs() + [pl.BlockSpec((_H3, 128), lambda i: (0, 0))],
        out_specs=pl.BlockSpec((_BL, 128), lambda i: (i, 0)),
        out_shape=jax.ShapeDtypeStruct((_N, 128), jnp.float32),
    )(hs, hs, w2)


def _final_body(sya_ref, syb_ref, hsa_ref, hsb_ref, dga_ref, dgb_ref,
                w2_ref, cb_ref, out_ref):
    di = _dinv(dga_ref, dgb_ref)
    z = (jnp.dot(di * _join(hsa_ref, hsb_ref), w2_ref[...],
                 preferred_element_type=jnp.float32)
         + di * (sya_ref[0] + syb_ref[0]) + cb_ref[...])
    m = jnp.max(z, axis=1, keepdims=True)
    e = jnp.exp(z - m)
    out_ref[...] = e / jnp.sum(e, axis=1, keepdims=True)


def _final(sy, hs, dga, dgb, w2, cb):
    grid = _N // _BL
    return pl.pallas_call(
        _final_body,
        grid=(grid,),
        in_specs=_acc_specs() + _acc_specs() + [
            pl.BlockSpec((_BL, 128), lambda i: (i, 0)),
            pl.BlockSpec((_BL, 128), lambda i: (i, 0)),
            pl.BlockSpec((_H3, 128), lambda i: (0, 0)),
            pl.BlockSpec((1, 128), lambda i: (0, 0))],
        out_specs=pl.BlockSpec((_BL, 128), lambda i: (i, 0)),
        out_shape=jax.ShapeDtypeStruct((_N, 128), jnp.float32),
    )(sy, sy, hs, hs, dga, dgb, w2, cb)


_NCHUNK = _EPT_SC // _CH
_NSTG = 4
_CPS = _NCHUNK // _NSTG
_JPS = _CPS // 2
_NCH_DEG = _EPT_DEG // _CH
_ZR = _RPT - 4 * _CH


def _sc_mesh():
    return plsc.VectorSubcoreMesh(core_axis_name="c", subcore_axis_name="s")


def _sc_degree(dst2d, zeros128, ones128):
    @functools.partial(
        pl.kernel,
        out_type=jax.ShapeDtypeStruct((_NC, _NPAD, 128), jnp.float32),
        mesh=_sc_mesh(),
        scratch_types=[
            pltpu.VMEM((_NCH_DEG, _CH), jnp.int32),
            pltpu.VMEM((_CH, 128), jnp.float32),
            pltpu.VMEM_SHARED((_NPAD, 128), jnp.float32),
        ],
    )
    def k(dst_hbm, zeros_hbm, ones_hbm, out_hbm, idx2d, ones_v, acc_sh):
        c = lax.axis_index("c")
        s = lax.axis_index("s")
        r0 = s * _RPT
        pltpu.sync_copy(zeros_hbm.at[pl.ds(r0, _RPT)],
                        acc_sh.at[pl.ds(r0, _RPT)])
        pltpu.sync_copy(ones_hbm, ones_v)
        row0 = (c * _NS + s) * _NCH_DEG
        pltpu.sync_copy(dst_hbm.at[pl.ds(row0, _NCH_DEG)], idx2d)
        plsc.subcore_barrier()

        def body(i, carry):
            pltpu.sync_copy(ones_v, acc_sh.at[idx2d.at[i]], add=True)
            return carry

        lax.fori_loop(0, _NCH_DEG, body, 0)
        plsc.subcore_barrier()
        pltpu.sync_copy(acc_sh.at[pl.ds(r0, _RPT)],
                        out_hbm.at[c].at[pl.ds(r0, _RPT)])

    return k(dst2d, zeros128, ones128)


def _sc_scatter(hs, src2d, dst2d, zeros128):
    @functools.partial(
        pl.kernel,
        out_type=jax.ShapeDtypeStruct((_NC, _NPAD, 128), jnp.float32),
        mesh=_sc_mesh(),
        scratch_types=[
            pltpu.VMEM((_CPS, _CH), jnp.int32),
            pltpu.VMEM((_CPS, _CH), jnp.int32),
            pltpu.VMEM((_CH, 128), jnp.float32),
            pltpu.VMEM((_CH, 128), jnp.float32),
            pltpu.VMEM_SHARED((_NPAD, 128), jnp.float32),
            pltpu.SemaphoreType.DMA,
            pltpu.SemaphoreType.DMA,
        ],
    )
    def k(hs_hbm, src_hbm, dst_hbm, zeros_hbm, out_hbm,
          src_st, dst_st, rows0, rows1, acc_sh, sem0, sem1):
        c = lax.axis_index("c")
        s = lax.axis_index("s")
        r0 = s * _RPT
        pltpu.sync_copy(zeros_hbm.at[pl.ds(r0, _RPT)],
                        acc_sh.at[pl.ds(r0, _RPT)])
        plsc.subcore_barrier()
        row0 = s * _NCHUNK
        half = hs_hbm.at[c]

        for stg in range(_NSTG):
            pltpu.sync_copy(src_hbm.at[pl.ds(row0 + stg * _CPS, _CPS)],
                            src_st)
            pltpu.sync_copy(dst_hbm.at[pl.ds(row0 + stg * _CPS, _CPS)],
                            dst_st)
            pltpu.async_copy(half.at[src_st.at[0]], rows0, sem0)

            def body(j, carry):
                i0 = j * 2
                pltpu.async_copy(half.at[src_st.at[i0 + 1]], rows1, sem1)
                pltpu.make_async_copy(
                    half.at[pl.ds(0, _CH)], rows0, sem0).wait()
                pltpu.sync_copy(rows0, acc_sh.at[dst_st.at[i0]], add=True)

                @pl.when(j < _JPS - 1)
                def _():
                    pltpu.async_copy(half.at[src_st.at[i0 + 2]], rows0, sem0)

                pltpu.make_async_copy(
                    half.at[pl.ds(0, _CH)], rows1, sem1).wait()
                pltpu.sync_copy(rows1, acc_sh.at[dst_st.at[i0 + 1]],
                                add=True)
                return carry

            lax.fori_loop(0, _JPS, body, 0)

        plsc.subcore_barrier()
        pltpu.sync_copy(acc_sh.at[pl.ds(r0, _RPT)],
                        out_hbm.at[c].at[pl.ds(r0, _RPT)])

    return k(hs, src2d, dst2d, zeros128)


_CPS2 = _NCH_DEG // 2
_JPS2 = _CPS2 // 2


def _sc_scatter_thin(y, src2d, dst2d, zeros128):
    @functools.partial(
        pl.kernel,
        out_type=jax.ShapeDtypeStruct((_NC, _NPAD, 128), jnp.float32),
        mesh=_sc_mesh(),
        scratch_types=[
            pltpu.VMEM((_CPS2, _CH), jnp.int32),
            pltpu.VMEM((_CPS2, _CH), jnp.int32),
            pltpu.VMEM((_CH, 128), jnp.float32),
            pltpu.VMEM((_CH, 128), jnp.float32),
            pltpu.VMEM_SHARED((_NPAD, 128), jnp.float32),
            pltpu.SemaphoreType.DMA,
            pltpu.SemaphoreType.DMA,
        ],
    )
    def k(y_hbm, src_hbm, dst_hbm, zeros_hbm, out_hbm,
          src_st, dst_st, rows0, rows1, acc_sh, sem0, sem1):
        c = lax.axis_index("c")
        s = lax.axis_index("s")
        r0 = s * _RPT
        pltpu.sync_copy(zeros_hbm.at[pl.ds(r0, _RPT)],
                        acc_sh.at[pl.ds(r0, _RPT)])
        plsc.subcore_barrier()
        base = (c * _NS + s) * _NCH_DEG

        for stg in range(2):
            pltpu.sync_copy(src_hbm.at[pl.ds(base + stg * _CPS2, _CPS2)],
                            src_st)
            pltpu.sync_copy(dst_hbm.at[pl.ds(base + stg * _CPS2, _CPS2)],
                            dst_st)
            pltpu.async_copy(y_hbm.at[src_st.at[0]], rows0, sem0)

            def body(j, carry):
                i0 = j * 2
                pltpu.async_copy(y_hbm.at[src_st.at[i0 + 1]], rows1, sem1)
                pltpu.make_async_copy(
                    y_hbm.at[pl.ds(0, _CH)], rows0, sem0).wait()
                pltpu.sync_copy(rows0, acc_sh.at[dst_st.at[i0]], add=True)

                @pl.when(j < _JPS2 - 1)
                def _():
                    pltpu.async_copy(y_hbm.at[src_st.at[i0 + 2]], rows0,
                                     sem0)

                pltpu.make_async_copy(
                    y_hbm.at[pl.ds(0, _CH)], rows1, sem1).wait()
                pltpu.sync_copy(rows1, acc_sh.at[dst_st.at[i0 + 1]],
                                add=True)
                return carry

            lax.fori_loop(0, _JPS2, body, 0)

        plsc.subcore_barrier()
        pltpu.sync_copy(acc_sh.at[pl.ds(r0, _RPT)],
                        out_hbm.at[c].at[pl.ds(r0, _RPT)])

    return k(y, src2d, dst2d, zeros128)


def _gru_prep(Wi, Wh, bi, bh):
    WiT = Wi.T
    WhT = Wh.T
    wr, wz, wn = WiT[:, :_H3], WiT[:, _H3:2 * _H3], WiT[:, 2 * _H3:]
    ur, uz, un = WhT[:, :_H3], WhT[:, _H3:2 * _H3], WhT[:, 2 * _H3:]
    br = (bi[:_H3] + bh[:_H3])[None, :]
    bz = (bi[_H3:2 * _H3] + bh[_H3:2 * _H3])[None, :]
    bn = bi[2 * _H3:][None, :]
    bhn = bh[2 * _H3:][None, :]
    return wr, wz, wn, ur, uz, un, br, bz, bn, bhn


def _pad_edges(ei):
    src, dst = ei[0], ei[1]
    npad = _EPAD - _E
    dpad = _N + (jnp.arange(npad, dtype=jnp.int32) % (_NPAD - _N))
    src_pad = jnp.concatenate([src, jnp.zeros((npad,), jnp.int32)])
    dst_pad = jnp.concatenate([dst, dpad])
    return (src_pad.reshape(_EPAD // _CH, _CH),
            dst_pad.reshape(_EPAD // _CH, _CH))


def _graph_side(xcat, R, bcat, ei, W0, b0, W1, b1, Wf, bf,
                zeros128, ones128):
    src2d, dst2d = _pad_edges(ei)
    degp = _sc_degree(dst2d, zeros128, ones128)
    dga, dgb = degp[0, :_N], degp[1, :_N]

    A = jnp.dot(R.T, W0.T)
    c = jnp.dot(bcat, W0.T)[None, :]
    hs0 = _layer0(xcat, A, c, dga, dgb)
    acc0 = _sc_scatter(hs0, src2d, dst2d, zeros128)[:, :_N]
    hs1 = _layer1(acc0, hs0, dga, dgb, b0[None, :], W1.T)

    w2 = jnp.zeros((_H3, 128), jnp.float32).at[:, :2].set(Wf.T)
    b2 = jnp.full((128,), -1e30, jnp.float32).at[:2].set(bf)
    cb = (jnp.dot(b1, w2) + b2)[None, :]
    y = _proj(hs1, w2)
    sy = _sc_scatter_thin(y, src2d, dst2d, zeros128)[:, :_N]
    out = _final(sy, hs1, dga, dgb, w2, cb)
    return out[:, :2]


def kernel(feature_1, feature_2_states, feature_3_states, feature_4,
           feature_5_states, feature_6, edge_index_1, edge_index_2, params):
    p = params
    fv1 = _gru(feature_1, *_gru_prep(p['gru1_Wi'], p['gru1_Wh'],
                                     p['gru1_bi'], p['gru1_bh']))
    fv6 = _gru(feature_6, *_gru_prep(p['gru2_Wi'], p['gru2_Wh'],
                                     p['gru2_bi'], p['gru2_bh']))

    z64 = jnp.zeros((64, _H3), jnp.float32)
    Rm = jnp.concatenate([
        jnp.concatenate([p['r1_W'], z64, z64], axis=1),
        jnp.concatenate([z64, p['r2_W'], z64], axis=1),
        jnp.concatenate([z64, z64, p['r3_W']], axis=1)], axis=0)
    bcm = jnp.concatenate([p['r1_b'], p['r2_b'], p['r3_b']])
    Xm = jnp.concatenate([fv1, feature_2_states, feature_3_states], axis=1)

    z200 = jnp.zeros((64, 200), jnp.float32)
    Rs = jnp.concatenate([
        jnp.concatenate([p['r4_W'], z64, z64], axis=1),
        jnp.concatenate([z200, p['r5_W'], z64], axis=1),
        jnp.concatenate([z200, z64, p['r6_W']], axis=1)], axis=0)
    bcs = jnp.concatenate([p['r4_b'], p['r5_b'], p['r6_b']])
    Xs = jnp.concatenate([feature_4, feature_5_states, fv6], axis=1)

    zeros128 = jnp.zeros((_NPAD, 128), jnp.float32)
    ones128 = jnp.ones((_CH, 128), jnp.float32)

    m_out = _graph_side(Xm, Rm, bcm, edge_index_1,
                        p['convm_W0'], p['convm_b0'],
                        p['convm_W1'], p['convm_b1'],
                        p['r7_W'], p['r7_b'], zeros128, ones128)
    s_out = _graph_side(Xs, Rs, bcs, edge_index_2,
                        p['convs_W0'], p['convs_b0'],
                        p['convs_W1'], p['convs_b1'],
                        p['r8_W'], p['r8_b'], zeros128, ones128)
    return (m_out, s_out)
